# Initial kernel scaffold; baseline (speedup 1.0000x reference)
import jax
import jax.numpy as jnp
from jax.experimental import pallas as pl


def kernel(x_user, x_item, edge_index):
    raise NotImplementedError("write your pallas kernel here")


# speedup vs baseline: 1.1007x; 1.1007x over previous
import functools

import jax
import jax.numpy as jnp
from jax import lax
from jax.experimental import pallas as pl
from jax.experimental.pallas import tpu as pltpu
from jax.experimental.pallas import tpu_sc as plsc

N_NODES = 10000
N_EDGES = 320000
D_FEAT = 128

NUM_CORES = 2
NUM_SUBCORES = 16
NUM_WORKERS = NUM_CORES * NUM_SUBCORES
EDGES_PER_WORKER = N_EDGES // NUM_WORKERS
CHUNK = 80
NUM_CHUNKS = EDGES_PER_WORKER // CHUNK
L = 16


def _sc_kernel_body(src_hbm, tgt_hbm, xu_hbm, xi_hbm, out_hbm,
                    idx_u, idx_i, rows_u, rows_i, out_v, sem_u, sem_i):
    wid = lax.axis_index("s") * NUM_CORES + lax.axis_index("c")
    base_w = wid * EDGES_PER_WORKER

    def chunk_body(c, carry):
        base = base_w + c * CHUNK
        pltpu.sync_copy(src_hbm.at[pl.ds(base, CHUNK)], idx_u)
        pltpu.sync_copy(tgt_hbm.at[pl.ds(base, CHUNK)], idx_i)
        cp_u = pltpu.async_copy(xu_hbm.at[idx_u], rows_u, sem_u)
        cp_i = pltpu.async_copy(xi_hbm.at[idx_i], rows_i, sem_i)
        cp_u.wait()
        cp_i.wait()

        lane = lax.broadcasted_iota(jnp.int32, (L,), 0)

        def group_body(g, carry2):
            row_idx = lane + g * L
            acc = jnp.zeros((L,), jnp.float32)
            for d in range(D_FEAT):
                col_idx = jnp.full((L,), d, jnp.int32)
                u = plsc.load_gather(rows_u, [row_idx, col_idx])
                v = plsc.load_gather(rows_i, [row_idx, col_idx])
                acc = acc + u * v
            out_v[pl.ds(g * L, L)] = 1.0 / (1.0 + jnp.exp(-acc))
            return carry2

        lax.fori_loop(0, CHUNK // L, group_body, 0)

        pltpu.sync_copy(out_v, out_hbm.at[pl.ds(base, CHUNK)])
        return carry

    lax.fori_loop(0, NUM_CHUNKS, chunk_body, 0)


@jax.jit
def _decode(x_user, x_item, src, tgt):
    mesh = plsc.VectorSubcoreMesh(
        core_axis_name="c", subcore_axis_name="s",
        num_cores=NUM_CORES, num_subcores=NUM_SUBCORES)
    run = pl.kernel(
        _sc_kernel_body,
        out_type=jax.ShapeDtypeStruct((N_EDGES,), jnp.float32),
        mesh=mesh,
        scratch_types=[
            pltpu.VMEM((CHUNK,), jnp.int32),
            pltpu.VMEM((CHUNK,), jnp.int32),
            pltpu.VMEM((CHUNK, D_FEAT), jnp.float32),
            pltpu.VMEM((CHUNK, D_FEAT), jnp.float32),
            pltpu.VMEM((CHUNK,), jnp.float32),
            pltpu.SemaphoreType.DMA,
            pltpu.SemaphoreType.DMA,
        ],
        compiler_params=pltpu.CompilerParams(needs_layout_passes=False),
    )
    return run(src, tgt, x_user, x_item)


def kernel(x_user, x_item, edge_index):
    src = edge_index[0].astype(jnp.int32)
    tgt = edge_index[1].astype(jnp.int32)
    return _decode(x_user, x_item, src, tgt)


# speedup vs baseline: 1.3675x; 1.2424x over previous
import functools

import jax
import jax.numpy as jnp
from jax import lax
from jax.experimental import pallas as pl
from jax.experimental.pallas import tpu as pltpu
from jax.experimental.pallas import tpu_sc as plsc

N_NODES = 10000
N_EDGES = 320000
D_FEAT = 128

NUM_CORES = 2
NUM_SUBCORES = 16
NUM_WORKERS = NUM_CORES * NUM_SUBCORES
EDGES_PER_WORKER = N_EDGES // NUM_WORKERS
CHUNK = 80
NBUF = 5
SUPER = CHUNK * NBUF
NUM_SUPER = EDGES_PER_WORKER // SUPER
L = 16


def _sc_kernel_body(src_hbm, tgt_hbm, xu_hbm, xi_hbm, out_hbm,
                    idx_u, idx_i, rows_u, rows_i, out_v, sem_u, sem_i):
    wid = lax.axis_index("s") * NUM_CORES + lax.axis_index("c")
    base_w = wid * EDGES_PER_WORKER

    lane = lax.broadcasted_iota(jnp.int32, (L,), 0)

    def compute_chunk(b):
        def group_body(g, carry):
            row_idx = lane + g * L
            zero = jnp.zeros((L,), jnp.float32)

            def d_body(d, accs):
                col = jnp.full((L,), d, jnp.int32)
                u = plsc.load_gather(rows_u.at[b], [row_idx, col])
                v = plsc.load_gather(rows_i.at[b], [row_idx, col])
                a0, a1, a2, a3 = accs
                return (a1, a2, a3, a0 + u * v)

            accs = lax.fori_loop(0, D_FEAT, d_body, (zero, zero, zero, zero),
                                 unroll=16)
            acc = (accs[0] + accs[1]) + (accs[2] + accs[3])
            out_v[pl.ds(b * CHUNK + g * L, L)] = 1.0 / (1.0 + jnp.exp(-acc))
            return carry

        lax.fori_loop(0, CHUNK // L, group_body, 0)

    @pl.loop(0, NUM_SUPER)
    def super_body(sc):
        base = sc * SUPER
        pltpu.sync_copy(src_hbm.at[pl.ds(base_w + base, SUPER)], idx_u)
        pltpu.sync_copy(tgt_hbm.at[pl.ds(base_w + base, SUPER)], idx_i)
        descs = []
        for b in range(NBUF):
            s = pl.ds(b * CHUNK, CHUNK)
            du = pltpu.async_copy(xu_hbm.at[idx_u.at[s]], rows_u.at[b], sem_u)
            di = pltpu.async_copy(xi_hbm.at[idx_i.at[s]], rows_i.at[b], sem_i)
            descs.append((du, di))
        for b in range(NBUF):
            du, di = descs[b]
            du.wait()
            di.wait()
            compute_chunk(b)
        pltpu.sync_copy(out_v, out_hbm.at[pl.ds(base_w + base, SUPER)])


@jax.jit
def _decode(x_user, x_item, src, tgt):
    mesh = plsc.VectorSubcoreMesh(
        core_axis_name="c", subcore_axis_name="s",
        num_cores=NUM_CORES, num_subcores=NUM_SUBCORES)
    run = pl.kernel(
        _sc_kernel_body,
        out_type=jax.ShapeDtypeStruct((N_EDGES,), jnp.float32),
        mesh=mesh,
        scratch_types=[
            pltpu.VMEM((SUPER,), jnp.int32),
            pltpu.VMEM((SUPER,), jnp.int32),
            pltpu.VMEM((NBUF, CHUNK, D_FEAT), jnp.float32),
            pltpu.VMEM((NBUF, CHUNK, D_FEAT), jnp.float32),
            pltpu.VMEM((SUPER,), jnp.float32),
            pltpu.SemaphoreType.DMA,
            pltpu.SemaphoreType.DMA,
        ],
        compiler_params=pltpu.CompilerParams(needs_layout_passes=False),
    )
    return run(src, tgt, x_user, x_item)


def kernel(x_user, x_item, edge_index):
    src = edge_index[0].astype(jnp.int32)
    tgt = edge_index[1].astype(jnp.int32)
    return _decode(x_user, x_item, src, tgt)


# speedup vs baseline: 1.3695x; 1.0014x over previous
import functools

import jax
import jax.numpy as jnp
from jax import lax
from jax.experimental import pallas as pl
from jax.experimental.pallas import tpu as pltpu
from jax.experimental.pallas import tpu_sc as plsc

N_NODES = 10000
N_EDGES = 320000
D_FEAT = 128

NUM_CORES = 2
NUM_SUBCORES = 16
NUM_WORKERS = NUM_CORES * NUM_SUBCORES
EDGES_PER_WORKER = N_EDGES // NUM_WORKERS
CHUNK = 80
NBUF = 5
SUPER = CHUNK * NBUF
NUM_SUPER = EDGES_PER_WORKER // SUPER
L = 16


def _sc_kernel_body(src_hbm, tgt_hbm, xu_hbm, xi_hbm, out_hbm,
                    idx_u, idx_i, rows_u, rows_i, out_v, sem_u, sem_i):
    wid = lax.axis_index("s") * NUM_CORES + lax.axis_index("c")
    base_w = wid * EDGES_PER_WORKER

    lane = lax.broadcasted_iota(jnp.int32, (L,), 0)

    def compute_chunk(b):
        @plsc.parallel_loop(0, CHUNK // L)
        def group_body(g):
            row_idx = lane + g * L
            zero = jnp.zeros((L,), jnp.float32)

            def d_body(d, accs):
                col = jnp.full((L,), d, jnp.int32)
                u = plsc.load_gather(rows_u.at[b], [row_idx, col])
                v = plsc.load_gather(rows_i.at[b], [row_idx, col])
                a0, a1, a2, a3 = accs
                return (a1, a2, a3, a0 + u * v)

            accs = lax.fori_loop(0, D_FEAT, d_body, (zero, zero, zero, zero),
                                 unroll=16)
            acc = (accs[0] + accs[1]) + (accs[2] + accs[3])
            out_v[pl.ds(b * CHUNK + g * L, L)] = 1.0 / (1.0 + jnp.exp(-acc))

    @pl.loop(0, NUM_SUPER)
    def super_body(sc):
        base = sc * SUPER
        pltpu.sync_copy(src_hbm.at[pl.ds(base_w + base, SUPER)], idx_u)
        pltpu.sync_copy(tgt_hbm.at[pl.ds(base_w + base, SUPER)], idx_i)
        descs = []
        for b in range(NBUF):
            s = pl.ds(b * CHUNK, CHUNK)
            du = pltpu.async_copy(xu_hbm.at[idx_u.at[s]], rows_u.at[b], sem_u)
            di = pltpu.async_copy(xi_hbm.at[idx_i.at[s]], rows_i.at[b], sem_i)
            descs.append((du, di))
        for b in range(NBUF):
            du, di = descs[b]
            du.wait()
            di.wait()
            compute_chunk(b)
        pltpu.sync_copy(out_v, out_hbm.at[pl.ds(base_w + base, SUPER)])


@jax.jit
def _decode(x_user, x_item, src, tgt):
    mesh = plsc.VectorSubcoreMesh(
        core_axis_name="c", subcore_axis_name="s",
        num_cores=NUM_CORES, num_subcores=NUM_SUBCORES)
    run = pl.kernel(
        _sc_kernel_body,
        out_type=jax.ShapeDtypeStruct((N_EDGES,), jnp.float32),
        mesh=mesh,
        scratch_types=[
            pltpu.VMEM((SUPER,), jnp.int32),
            pltpu.VMEM((SUPER,), jnp.int32),
            pltpu.VMEM((NBUF, CHUNK, D_FEAT), jnp.float32),
            pltpu.VMEM((NBUF, CHUNK, D_FEAT), jnp.float32),
            pltpu.VMEM((SUPER,), jnp.float32),
            pltpu.SemaphoreType.DMA,
            pltpu.SemaphoreType.DMA,
        ],
        compiler_params=pltpu.CompilerParams(needs_layout_passes=False),
    )
    return run(src, tgt, x_user, x_item)


def kernel(x_user, x_item, edge_index):
    src = edge_index[0].astype(jnp.int32)
    tgt = edge_index[1].astype(jnp.int32)
    return _decode(x_user, x_item, src, tgt)


# speedup vs baseline: 3.5837x; 2.6169x over previous
import functools

import jax
import jax.numpy as jnp
from jax import lax
from jax.experimental import pallas as pl
from jax.experimental.pallas import tpu as pltpu
from jax.experimental.pallas import tpu_sc as plsc

N_NODES = 10000
N_EDGES = 320000
D_FEAT = 128

NUM_CORES = 2
NUM_SUBCORES = 16
NUM_WORKERS = NUM_CORES * NUM_SUBCORES
EDGES_PER_WORKER = N_EDGES // NUM_WORKERS
CHUNK = 80
NBUF = 5
SUPER = CHUNK * NBUF
NUM_SUPER = EDGES_PER_WORKER // SUPER
L = 16


def _sc_kernel_body(src_hbm, tgt_hbm, xu_hbm, xi_hbm, out_hbm,
                    idx_u, idx_i, rows_u, rows_i, out_v, sem_u, sem_i):
    wid = lax.axis_index("s") * NUM_CORES + lax.axis_index("c")
    base_w = wid * EDGES_PER_WORKER

    lane = lax.broadcasted_iota(jnp.int32, (L,), 0)

    def compute_chunk(b):
        def group_body(g, carry):
            res = jnp.zeros((L,), jnp.float32)
            for e in range(L):
                row = g * L + e
                ps = [rows_u.at[b][row, pl.ds(k * L, L)]
                      * rows_i.at[b][row, pl.ds(k * L, L)]
                      for k in range(D_FEAT // L)]
                p = ((ps[0] + ps[1]) + (ps[2] + ps[3])) \
                    + ((ps[4] + ps[5]) + (ps[6] + ps[7]))
                res = jnp.where(lane == e, jnp.sum(p), res)
            out_v[pl.ds(b * CHUNK + g * L, L)] = 1.0 / (1.0 + jnp.exp(-res))
            return carry

        lax.fori_loop(0, CHUNK // L, group_body, 0)

    @pl.loop(0, NUM_SUPER)
    def super_body(sc):
        base = sc * SUPER
        pltpu.sync_copy(src_hbm.at[pl.ds(base_w + base, SUPER)], idx_u)
        pltpu.sync_copy(tgt_hbm.at[pl.ds(base_w + base, SUPER)], idx_i)
        descs = []
        for b in range(NBUF):
            s = pl.ds(b * CHUNK, CHUNK)
            du = pltpu.async_copy(xu_hbm.at[idx_u.at[s]], rows_u.at[b], sem_u)
            di = pltpu.async_copy(xi_hbm.at[idx_i.at[s]], rows_i.at[b], sem_i)
            descs.append((du, di))
        for b in range(NBUF):
            du, di = descs[b]
            du.wait()
            di.wait()
            compute_chunk(b)
        pltpu.sync_copy(out_v, out_hbm.at[pl.ds(base_w + base, SUPER)])


@jax.jit
def _decode(x_user, x_item, src, tgt):
    mesh = plsc.VectorSubcoreMesh(
        core_axis_name="c", subcore_axis_name="s",
        num_cores=NUM_CORES, num_subcores=NUM_SUBCORES)
    run = pl.kernel(
        _sc_kernel_body,
        out_type=jax.ShapeDtypeStruct((N_EDGES,), jnp.float32),
        mesh=mesh,
        scratch_types=[
            pltpu.VMEM((SUPER,), jnp.int32),
            pltpu.VMEM((SUPER,), jnp.int32),
            pltpu.VMEM((NBUF, CHUNK, D_FEAT), jnp.float32),
            pltpu.VMEM((NBUF, CHUNK, D_FEAT), jnp.float32),
            pltpu.VMEM((SUPER,), jnp.float32),
            pltpu.SemaphoreType.DMA,
            pltpu.SemaphoreType.DMA,
        ],
        compiler_params=pltpu.CompilerParams(needs_layout_passes=False),
    )
    return run(src, tgt, x_user, x_item)


def kernel(x_user, x_item, edge_index):
    src = edge_index[0].astype(jnp.int32)
    tgt = edge_index[1].astype(jnp.int32)
    return _decode(x_user, x_item, src, tgt)


# speedup vs baseline: 3.8537x; 1.0753x over previous
import functools

import jax
import jax.numpy as jnp
from jax import lax
from jax.experimental import pallas as pl
from jax.experimental.pallas import tpu as pltpu
from jax.experimental.pallas import tpu_sc as plsc

N_NODES = 10000
N_EDGES = 320000
D_FEAT = 128

NUM_CORES = 2
NUM_SUBCORES = 16
NUM_WORKERS = NUM_CORES * NUM_SUBCORES
EDGES_PER_WORKER = N_EDGES // NUM_WORKERS
CHUNK = 80
NBUF = 5
SUPER = CHUNK * NBUF
NUM_SUPER = EDGES_PER_WORKER // SUPER
L = 16


def _sc_kernel_body(src_hbm, tgt_hbm, xu_hbm, xi_hbm, out_hbm,
                    idx_u, idx_i, rows_u, rows_i, out_v, sem_u, sem_i, sem_x):
    wid = lax.axis_index("s") * NUM_CORES + lax.axis_index("c")
    base_w = wid * EDGES_PER_WORKER

    lane = lax.broadcasted_iota(jnp.int32, (L,), 0)
    perms = [lax.bitwise_xor(lane, jnp.int32(st)) for st in (8, 4, 2, 1)]

    def compute_chunk(base, b):
        def group_body(g, carry):
            res = jnp.zeros((L,), jnp.float32)
            for e in range(L):
                row = g * L + e
                ps = [rows_u.at[b][row, pl.ds(k * L, L)]
                      * rows_i.at[b][row, pl.ds(k * L, L)]
                      for k in range(D_FEAT // L)]
                p = ((ps[0] + ps[1]) + (ps[2] + ps[3])) \
                    + ((ps[4] + ps[5]) + (ps[6] + ps[7]))
                for perm in perms:
                    p = p + p[perm]
                res = res + jnp.where(lane == e, p, 0.0)
            out_v[pl.ds(b * CHUNK + g * L, L)] = 1.0 / (1.0 + jnp.exp(-res))
            return carry

        lax.fori_loop(0, CHUNK // L, group_body, 0)

    @pl.loop(0, NUM_SUPER)
    def super_body(sc):
        base = sc * SUPER
        diu = pltpu.async_copy(src_hbm.at[pl.ds(base_w + base, SUPER)], idx_u, sem_x)
        dii = pltpu.async_copy(tgt_hbm.at[pl.ds(base_w + base, SUPER)], idx_i, sem_x)
        diu.wait()
        dii.wait()
        descs = []
        for b in range(NBUF):
            s = pl.ds(b * CHUNK, CHUNK)
            du = pltpu.async_copy(xu_hbm.at[idx_u.at[s]], rows_u.at[b], sem_u)
            di = pltpu.async_copy(xi_hbm.at[idx_i.at[s]], rows_i.at[b], sem_i)
            descs.append((du, di))
        for b in range(NBUF):
            du, di = descs[b]
            du.wait()
            di.wait()
            compute_chunk(base, b)
        pltpu.sync_copy(out_v, out_hbm.at[pl.ds(base_w + base, SUPER)])


@jax.jit
def _decode(x_user, x_item, src, tgt):
    mesh = plsc.VectorSubcoreMesh(
        core_axis_name="c", subcore_axis_name="s",
        num_cores=NUM_CORES, num_subcores=NUM_SUBCORES)
    run = pl.kernel(
        _sc_kernel_body,
        out_type=jax.ShapeDtypeStruct((N_EDGES,), jnp.float32),
        mesh=mesh,
        scratch_types=[
            pltpu.VMEM((SUPER,), jnp.int32),
            pltpu.VMEM((SUPER,), jnp.int32),
            pltpu.VMEM((NBUF, CHUNK, D_FEAT), jnp.float32),
            pltpu.VMEM((NBUF, CHUNK, D_FEAT), jnp.float32),
            pltpu.VMEM((SUPER,), jnp.float32),
            pltpu.SemaphoreType.DMA,
            pltpu.SemaphoreType.DMA,
            pltpu.SemaphoreType.DMA,
        ],
        compiler_params=pltpu.CompilerParams(needs_layout_passes=False),
    )
    return run(src, tgt, x_user, x_item)


def kernel(x_user, x_item, edge_index):
    src = edge_index[0].astype(jnp.int32)
    tgt = edge_index[1].astype(jnp.int32)
    return _decode(x_user, x_item, src, tgt)
